# Initial kernel scaffold; baseline (speedup 1.0000x reference)
import jax
import jax.numpy as jnp
from jax.experimental import pallas as pl


def kernel(points, faces, camera_rot, camera_pos, camera_proj, uv, texture, ts):
    raise NotImplementedError("write your pallas kernel here")


# speedup vs baseline: 1.4010x; 1.4010x over previous
import functools

import jax
import jax.numpy as jnp
from jax.experimental import pallas as pl

H = 512
W = 512
HW = H * W


def _composite_body(ims_ref, mrep_ref, probs_ref, masks_ref, ren_ref, prob_ref, fg_ref):
    ren = ims_ref[0]
    prob = probs_ref[0]
    fg = masks_ref[0]
    for i in range(1, 4):
        mrep = mrep_ref[i] > 0.5
        m = masks_ref[i] > 0.5
        ren = jnp.where(mrep, ims_ref[i], ren)
        prob = jnp.where(m, probs_ref[i], prob)
        fg = jnp.where(m, masks_ref[i], fg)
    ren_ref[...] = ren
    prob_ref[...] = prob
    fg_ref[...] = fg


@jax.jit
def _composite(ims_flat, maskrep, probs, masks):
    BR = 64
    grid = (H // BR,)
    return pl.pallas_call(
        _composite_body,
        grid=grid,
        in_specs=[
            pl.BlockSpec((4, BR, 3 * W), lambda r: (0, r, 0)),
            pl.BlockSpec((4, BR, 3 * W), lambda r: (0, r, 0)),
            pl.BlockSpec((4, BR, W), lambda r: (0, r, 0)),
            pl.BlockSpec((4, BR, W), lambda r: (0, r, 0)),
        ],
        out_specs=[
            pl.BlockSpec((BR, 3 * W), lambda r: (r, 0)),
            pl.BlockSpec((BR, W), lambda r: (r, 0)),
            pl.BlockSpec((BR, W), lambda r: (r, 0)),
        ],
        out_shape=[
            jax.ShapeDtypeStruct((H, 3 * W), jnp.float32),
            jax.ShapeDtypeStruct((H, W), jnp.float32),
            jax.ShapeDtypeStruct((H, W), jnp.float32),
        ],
    )(ims_flat, maskrep, probs, masks)


def kernel(points, faces, camera_rot, camera_pos, camera_proj, uv, texture, ts):
    b = points.shape[0]
    F = faces.shape[0]
    dist_inds = jnp.argsort(ts[:, 2])[::-1]

    pc = jnp.einsum('bij,bpj->bpi', camera_rot, points - camera_pos[:, None, :])
    z = pc[:, :, 2] + 10.0
    x2 = pc[:, :, 0] * camera_proj[0, 0] / z
    y2 = pc[:, :, 1] * camera_proj[1, 0] / z

    f0, f1, f2 = faces[:, 0], faces[:, 1], faces[:, 2]
    v0 = pc[:, f0, :]
    v1 = pc[:, f1, :]
    v2 = pc[:, f2, :]
    normal = jnp.cross(v1 - v0, v2 - v0)
    nz = normal[:, :, 2]
    n1 = normal / (jnp.linalg.norm(normal, axis=2, keepdims=True) + 1e-8)
    n1_list = [n1[i][None] for i in range(b)]

    cx = (x2[:, f0] + x2[:, f1] + x2[:, f2]) / 3.0
    cy = (y2[:, f0] + y2[:, f1] + y2[:, f2]) / 3.0
    depth = (v0[:, :, 2] + v1[:, :, 2] + v2[:, :, 2]) / 3.0
    inb = (cx >= -1.0) & (cx <= 1.0) & (cy >= -1.0) & (cy <= 1.0)
    valid = inb & (nz > 0.0)
    px = jnp.clip(jnp.round((cx + 1.0) * 0.5 * (W - 1)), 0, W - 1).astype(jnp.int32)
    py = jnp.clip(jnp.round((1.0 - cy) * 0.5 * (H - 1)), 0, H - 1).astype(jnp.int32)
    flat = py * W + px
    zkey = jnp.where(valid, -depth, -jnp.inf)

    featu = (uv[:, f0, 0] + uv[:, f1, 0] + uv[:, f2, 0]) / 3.0
    featv = (uv[:, f0, 1] + uv[:, f1, 1] + uv[:, f2, 1]) / 3.0

    def raster_one(flat_i, zkey_i, valid_i, featu_i, featv_i):
        zbuf = jnp.full((HW,), -jnp.inf, dtype=jnp.float32).at[flat_i].max(zkey_i)
        win = (valid_i & (zkey_i >= zbuf[flat_i] - 1e-6)).astype(jnp.float32)
        imu = jnp.zeros((HW,), jnp.float32).at[flat_i].add(featu_i * win)
        imv = jnp.zeros((HW,), jnp.float32).at[flat_i].add(featv_i * win)
        cnt = jnp.zeros((HW,), jnp.float32).at[flat_i].add(win)
        return imu, imv, cnt

    imu, imv, cnt = jax.vmap(raster_one)(flat, zkey, valid, featu, featv)

    u = jnp.clip(imu, 0.0, 1.0)
    v = jnp.clip(imv, 0.0, 1.0)
    tw = texture.shape[3]
    th = texture.shape[2]
    tx = jnp.round(u * (tw - 1)).astype(jnp.int32)
    ty = jnp.round((1.0 - v) * (th - 1)).astype(jnp.int32)
    tflat = ty * tw + tx

    def shade_one(tex_i, tflat_i, cnt_i):
        texf = tex_i.reshape(3, th * tw)
        col = texf[:, tflat_i]
        return col * cnt_i[None, :]

    colors = jax.vmap(shade_one)(texture, tflat, cnt)

    ims = colors[dist_inds]
    masks = cnt[dist_inds]
    probs = jnp.clip(cnt, 0.0, 1.0)[dist_inds]
    ims_flat = ims.transpose(0, 2, 1).reshape(4, H, 3 * W)
    maskrep = jnp.repeat(masks[:, :, None], 3, axis=2).reshape(4, H, 3 * W)
    ren, prob, fg = _composite(ims_flat, maskrep, probs.reshape(4, H, W),
                               masks.reshape(4, H, W))

    imrender = ren.reshape(H, W, 3)[None]
    improb = prob.reshape(H, W, 1)[None]
    fg_mask = fg.reshape(H, W, 1)[None]
    return (imrender, improb, n1_list, fg_mask)


# speedup vs baseline: 3.8618x; 2.7565x over previous
import functools

import jax
import jax.numpy as jnp
from jax import lax
from jax.experimental import pallas as pl
from jax.experimental.pallas import tpu as pltpu
from jax.experimental.pallas import tpu_sc as plsc

H = 512
W = 512
HW = H * W
TW = 1024
TH = 1024
TEXN = 3 * TH * TW
NW = 32
PPW = HW // NW
PWIN = 4096
NWIN = PPW // PWIN
TWO23 = 8388608.0


def _rte(x):
    return (x + TWO23) - TWO23


def _frag_body(imgs, tex, bases, ren_o, prob_o, fg_o,
               bas_v, bufs, tidx0, tidx1, tidx2, rv, gv, bv,
               outr, outg, outb, outp, outf, sem):
    wid = lax.axis_index("s") * 2 + lax.axis_index("c")
    lanes = lax.iota(jnp.int32, 16)
    pltpu.sync_copy(bases, bas_v)

    for win in range(NWIN):
        pix0 = wid * PPW + win * PWIN
        for t in range(3):
            for j in range(4):
                pltpu.sync_copy(imgs.at[pl.ds((t * 4 + j) * HW + pix0, PWIN)],
                                bufs[t * 4 + j])

        def it_body(i, _):
            sl = pl.ds(i * 16, 16)
            cnt = [bufs[8 + j][sl] for j in range(4)]
            u = bufs[0][sl]
            v = bufs[4][sl]
            cw = cnt[0]
            base = bas_v[pl.ds(0, 16)]
            for j in range(1, 4):
                m = cnt[j] > 0.5
                u = jnp.where(m, bufs[j][sl], u)
                v = jnp.where(m, bufs[4 + j][sl], v)
                cw = jnp.where(m, cnt[j], cw)
                base = jnp.where(m, bas_v[pl.ds(j * 16, 16)], base)
            uu = jnp.minimum(jnp.maximum(u, 0.0), 1.0)
            vv = jnp.minimum(jnp.maximum(v, 0.0), 1.0)
            tx = _rte(uu * float(TW - 1)).astype(jnp.int32)
            ty = _rte((1.0 - vv) * float(TH - 1)).astype(jnp.int32)
            tidx = base + ty * TW + tx
            covered = cw > 0.5
            spread = (pix0 + i * 16 + lanes) & (TEXN - 1)
            tidx = jnp.where(covered, tidx, spread)
            tidx0[sl] = tidx
            tidx1[sl] = tidx + TH * TW
            tidx2[sl] = tidx + 2 * TH * TW
            outf[sl] = cw
            outp[sl] = jnp.minimum(cw, 1.0)
            return _

        lax.fori_loop(0, PWIN // 16, it_body, None)

        pltpu.async_copy(tex.at[tidx0], rv, sem).wait()
        pltpu.async_copy(tex.at[tidx1], gv, sem).wait()
        pltpu.async_copy(tex.at[tidx2], bv, sem).wait()

        def mul_body(i, _):
            sl = pl.ds(i * 16, 16)
            cw = outf[sl]
            outr[sl] = rv[sl] * cw
            outg[sl] = gv[sl] * cw
            outb[sl] = bv[sl] * cw
            return _

        lax.fori_loop(0, PWIN // 16, mul_body, None)

        pltpu.sync_copy(outr, ren_o.at[pl.ds(pix0, PWIN)])
        pltpu.sync_copy(outg, ren_o.at[pl.ds(HW + pix0, PWIN)])
        pltpu.sync_copy(outb, ren_o.at[pl.ds(2 * HW + pix0, PWIN)])
        pltpu.sync_copy(outp, prob_o.at[pl.ds(pix0, PWIN)])
        pltpu.sync_copy(outf, fg_o.at[pl.ds(pix0, PWIN)])


def _frag_body_wrap(imgs, tex, bases, ren_o, prob_o, fg_o, *scratch):
    bufs = list(scratch[1:13])
    _frag_body(imgs, tex, bases, ren_o, prob_o, fg_o,
               scratch[0], bufs, *scratch[13:])


@jax.jit
def _frag_sc(imgs, tex, bases):
    f32 = jnp.float32
    i32 = jnp.int32
    k = pl.kernel(
        _frag_body_wrap,
        mesh=plsc.VectorSubcoreMesh(core_axis_name="c", subcore_axis_name="s"),
        out_type=[
            jax.ShapeDtypeStruct((3 * HW,), f32),
            jax.ShapeDtypeStruct((HW,), f32),
            jax.ShapeDtypeStruct((HW,), f32),
        ],
        scratch_types=[pltpu.VMEM((64,), i32)]
        + [pltpu.VMEM((PWIN,), f32) for _ in range(12)]
        + [pltpu.VMEM((PWIN,), i32) for _ in range(3)]
        + [pltpu.VMEM((PWIN,), f32) for _ in range(8)]
        + [pltpu.SemaphoreType.DMA],
    )
    return k(imgs, tex, bases)


def kernel(points, faces, camera_rot, camera_pos, camera_proj, uv, texture, ts):
    b = points.shape[0]
    F = faces.shape[0]
    dist_inds = jnp.argsort(ts[:, 2])[::-1]

    pc = jnp.einsum('bij,bpj->bpi', camera_rot, points - camera_pos[:, None, :])
    z = pc[:, :, 2] + 10.0
    x2 = pc[:, :, 0] * camera_proj[0, 0] / z
    y2 = pc[:, :, 1] * camera_proj[1, 0] / z

    f0, f1, f2 = faces[:, 0], faces[:, 1], faces[:, 2]
    v0 = pc[:, f0, :]
    v1 = pc[:, f1, :]
    v2 = pc[:, f2, :]
    normal = jnp.cross(v1 - v0, v2 - v0)
    nz = normal[:, :, 2]
    n1 = normal / (jnp.linalg.norm(normal, axis=2, keepdims=True) + 1e-8)
    n1_list = [n1[i][None] for i in range(b)]

    cx = (x2[:, f0] + x2[:, f1] + x2[:, f2]) / 3.0
    cy = (y2[:, f0] + y2[:, f1] + y2[:, f2]) / 3.0
    depth = (v0[:, :, 2] + v1[:, :, 2] + v2[:, :, 2]) / 3.0
    inb = (cx >= -1.0) & (cx <= 1.0) & (cy >= -1.0) & (cy <= 1.0)
    valid = inb & (nz > 0.0)
    px = jnp.clip(jnp.round((cx + 1.0) * 0.5 * (W - 1)), 0, W - 1).astype(jnp.int32)
    py = jnp.clip(jnp.round((1.0 - cy) * 0.5 * (H - 1)), 0, H - 1).astype(jnp.int32)
    flat = py * W + px
    zkey = jnp.where(valid, -depth, -jnp.inf)

    featu = (uv[:, f0, 0] + uv[:, f1, 0] + uv[:, f2, 0]) / 3.0
    featv = (uv[:, f0, 1] + uv[:, f1, 1] + uv[:, f2, 1]) / 3.0

    def raster_one(flat_i, zkey_i, valid_i, featu_i, featv_i):
        zbuf = jnp.full((HW,), -jnp.inf, dtype=jnp.float32).at[flat_i].max(zkey_i)
        win = (valid_i & (zkey_i >= zbuf[flat_i] - 1e-6)).astype(jnp.float32)
        imu = jnp.zeros((HW,), jnp.float32).at[flat_i].add(featu_i * win)
        imv = jnp.zeros((HW,), jnp.float32).at[flat_i].add(featv_i * win)
        cnt = jnp.zeros((HW,), jnp.float32).at[flat_i].add(win)
        return imu, imv, cnt

    imu, imv, cnt = jax.vmap(raster_one)(flat, zkey, valid, featu, featv)

    imgs = jnp.stack([imu[dist_inds], imv[dist_inds], cnt[dist_inds]]).reshape(-1)
    bases = jnp.broadcast_to((dist_inds.astype(jnp.int32) * TEXN)[:, None],
                             (4, 16)).reshape(-1)
    ren, prob, fg = _frag_sc(imgs, texture.reshape(-1), bases)

    imrender = ren.reshape(3, H, W).transpose(1, 2, 0)[None]
    improb = prob.reshape(H, W, 1)[None]
    fg_mask = fg.reshape(H, W, 1)[None]
    return (imrender, improb, n1_list, fg_mask)


# speedup vs baseline: 4.9115x; 1.2718x over previous
import functools

import jax
import jax.numpy as jnp
from jax import lax
from jax.experimental import pallas as pl
from jax.experimental.pallas import tpu as pltpu
from jax.experimental.pallas import tpu_sc as plsc

H = 512
W = 512
HW = H * W
TW = 1024
TH = 1024
TEXN = 3 * TH * TW
NW = 32
PPW = HW // NW
PWIN = 4096
NWIN = PPW // PWIN
TWO23 = 8388608.0


def _rte(x):
    return (x + TWO23) - TWO23


F_PAD = 50176
FW = 1568
NFW = F_PAD // FW
SEG = HW // 8
NEG_INF = float("-inf")


def _accum_body(flat_h, zkey_h, fu_h, fv_h, zb_h, imgs_o,
                shared, flv, zkv, fuv, fvv, zidx, zbv, au, av, ac,
                p0, p1, p2, zv, sem):
    c = lax.axis_index("c")
    s = lax.axis_index("s")
    Z = HW // 16

    def z_body(i, _):
        zv[pl.ds(i * 16, 16)] = jnp.zeros((16,), jnp.float32)
        return _

    lax.fori_loop(0, 4096 // 16, z_body, None)
    for p in range(6):
        for kq in range(Z // 4096):
            pltpu.sync_copy(zv, shared.at[pl.ds(p * HW + s * Z + kq * 4096, 4096)])
    plsc.subcore_barrier()

    for lb in range(2):
        bb = lb * 2 + c
        for wi in range(2):
            fb = bb * F_PAD + (s * 2 + wi) * FW
            pltpu.sync_copy(flat_h.at[pl.ds(fb, FW)], flv)
            pltpu.sync_copy(zkey_h.at[pl.ds(fb, FW)], zkv)
            pltpu.sync_copy(fu_h.at[pl.ds(fb, FW)], fuv)
            pltpu.sync_copy(fv_h.at[pl.ds(fb, FW)], fvv)

            def ix_body(i, _):
                sl = pl.ds(i * 16, 16)
                zidx[sl] = bb * HW + flv[sl]
                return _

            lax.fori_loop(0, FW // 16, ix_body, None)
            pltpu.async_copy(zb_h.at[zidx], zbv, sem).wait()

            def acc_body(i, _):
                sl = pl.ds(i * 16, 16)
                fl = flv[sl]
                zk = zkv[sl]
                win = (zk > NEG_INF) & (zk >= zbv[sl] - 1e-6)
                w1 = jnp.where(win, 1.0, 0.0)
                au[sl] = fuv[sl] * w1
                av[sl] = fvv[sl] * w1
                ac[sl] = w1
                base0 = lb * 3 * HW + fl
                p0[sl] = base0
                p1[sl] = base0 + HW
                p2[sl] = base0 + 2 * HW
                return _

            lax.fori_loop(0, FW // 16, acc_body, None)
            pltpu.sync_copy(au, shared.at[p0], add=True)
            pltpu.sync_copy(av, shared.at[p1], add=True)
            pltpu.sync_copy(ac, shared.at[p2], add=True)

    plsc.subcore_barrier()
    for p in range(6):
        lb, t = p // 3, p % 3
        bb = lb * 2 + c
        pltpu.sync_copy(shared.at[pl.ds(p * HW + s * Z, Z)],
                        imgs_o.at[pl.ds((t * 4 + bb) * HW + s * Z, Z)])


@jax.jit
def _accum_sc(flat_h, zkey_h, fu_h, fv_h, zb_h):
    f32 = jnp.float32
    i32 = jnp.int32
    k = pl.kernel(
        _accum_body,
        mesh=plsc.VectorSubcoreMesh(core_axis_name="c", subcore_axis_name="s"),
        out_type=[jax.ShapeDtypeStruct((12 * HW,), f32)],
        scratch_types=[
            pltpu.VMEM_SHARED((6 * HW,), f32),
            pltpu.VMEM((FW,), i32),
            pltpu.VMEM((FW,), f32),
            pltpu.VMEM((FW,), f32),
            pltpu.VMEM((FW,), f32),
            pltpu.VMEM((FW,), i32),
            pltpu.VMEM((FW,), f32),
            pltpu.VMEM((FW,), f32),
            pltpu.VMEM((FW,), f32),
            pltpu.VMEM((FW,), f32),
            pltpu.VMEM((FW,), i32),
            pltpu.VMEM((FW,), i32),
            pltpu.VMEM((FW,), i32),
            pltpu.VMEM((4096,), f32),
            pltpu.SemaphoreType.DMA,
        ],
    )
    return k(flat_h, zkey_h, fu_h, fv_h, zb_h)


def _frag_body(imgs, tex, bases, ren_o, prob_o, fg_o,
               bas_v, bufs, tidx0, tidx1, tidx2, rv, gv, bv,
               outr, outg, outb, outp, outf, sem):
    wid = lax.axis_index("s") * 2 + lax.axis_index("c")
    lanes = lax.iota(jnp.int32, 16)
    pltpu.sync_copy(bases, bas_v)

    for win in range(NWIN):
        pix0 = wid * PPW + win * PWIN
        for t in range(3):
            for j in range(4):
                pltpu.sync_copy(imgs.at[pl.ds((t * 4 + j) * HW + pix0, PWIN)],
                                bufs[t * 4 + j])

        def it_body(i, _):
            sl = pl.ds(i * 16, 16)
            cnt = [bufs[8 + j][sl] for j in range(4)]
            u = bufs[0][sl]
            v = bufs[4][sl]
            cw = cnt[0]
            base = bas_v[pl.ds(0, 16)]
            for j in range(1, 4):
                m = cnt[j] > 0.5
                u = jnp.where(m, bufs[j][sl], u)
                v = jnp.where(m, bufs[4 + j][sl], v)
                cw = jnp.where(m, cnt[j], cw)
                base = jnp.where(m, bas_v[pl.ds(j * 16, 16)], base)
            uu = jnp.minimum(jnp.maximum(u, 0.0), 1.0)
            vv = jnp.minimum(jnp.maximum(v, 0.0), 1.0)
            tx = _rte(uu * float(TW - 1)).astype(jnp.int32)
            ty = _rte((1.0 - vv) * float(TH - 1)).astype(jnp.int32)
            tidx = base + ty * TW + tx
            covered = cw > 0.5
            spread = (pix0 + i * 16 + lanes) & (TEXN - 1)
            tidx = jnp.where(covered, tidx, spread)
            tidx0[sl] = tidx
            tidx1[sl] = tidx + TH * TW
            tidx2[sl] = tidx + 2 * TH * TW
            outf[sl] = cw
            outp[sl] = jnp.minimum(cw, 1.0)
            return _

        lax.fori_loop(0, PWIN // 16, it_body, None)

        pltpu.async_copy(tex.at[tidx0], rv, sem).wait()
        pltpu.async_copy(tex.at[tidx1], gv, sem).wait()
        pltpu.async_copy(tex.at[tidx2], bv, sem).wait()

        def mul_body(i, _):
            sl = pl.ds(i * 16, 16)
            cw = outf[sl]
            outr[sl] = rv[sl] * cw
            outg[sl] = gv[sl] * cw
            outb[sl] = bv[sl] * cw
            return _

        lax.fori_loop(0, PWIN // 16, mul_body, None)

        pltpu.sync_copy(outr, ren_o.at[pl.ds(pix0, PWIN)])
        pltpu.sync_copy(outg, ren_o.at[pl.ds(HW + pix0, PWIN)])
        pltpu.sync_copy(outb, ren_o.at[pl.ds(2 * HW + pix0, PWIN)])
        pltpu.sync_copy(outp, prob_o.at[pl.ds(pix0, PWIN)])
        pltpu.sync_copy(outf, fg_o.at[pl.ds(pix0, PWIN)])


def _frag_body_wrap(imgs, tex, bases, ren_o, prob_o, fg_o, *scratch):
    bufs = list(scratch[1:13])
    _frag_body(imgs, tex, bases, ren_o, prob_o, fg_o,
               scratch[0], bufs, *scratch[13:])


@jax.jit
def _frag_sc(imgs, tex, bases):
    f32 = jnp.float32
    i32 = jnp.int32
    k = pl.kernel(
        _frag_body_wrap,
        mesh=plsc.VectorSubcoreMesh(core_axis_name="c", subcore_axis_name="s"),
        out_type=[
            jax.ShapeDtypeStruct((3 * HW,), f32),
            jax.ShapeDtypeStruct((HW,), f32),
            jax.ShapeDtypeStruct((HW,), f32),
        ],
        scratch_types=[pltpu.VMEM((64,), i32)]
        + [pltpu.VMEM((PWIN,), f32) for _ in range(12)]
        + [pltpu.VMEM((PWIN,), i32) for _ in range(3)]
        + [pltpu.VMEM((PWIN,), f32) for _ in range(8)]
        + [pltpu.SemaphoreType.DMA],
    )
    return k(imgs, tex, bases)


def kernel(points, faces, camera_rot, camera_pos, camera_proj, uv, texture, ts):
    b = points.shape[0]
    F = faces.shape[0]
    dist_inds = jnp.argsort(ts[:, 2])[::-1]

    pc = jnp.einsum('bij,bpj->bpi', camera_rot, points - camera_pos[:, None, :])
    z = pc[:, :, 2] + 10.0
    x2 = pc[:, :, 0] * camera_proj[0, 0] / z
    y2 = pc[:, :, 1] * camera_proj[1, 0] / z

    f0, f1, f2 = faces[:, 0], faces[:, 1], faces[:, 2]
    v0 = pc[:, f0, :]
    v1 = pc[:, f1, :]
    v2 = pc[:, f2, :]
    normal = jnp.cross(v1 - v0, v2 - v0)
    nz = normal[:, :, 2]
    n1 = normal / (jnp.linalg.norm(normal, axis=2, keepdims=True) + 1e-8)
    n1_list = [n1[i][None] for i in range(b)]

    cx = (x2[:, f0] + x2[:, f1] + x2[:, f2]) / 3.0
    cy = (y2[:, f0] + y2[:, f1] + y2[:, f2]) / 3.0
    depth = (v0[:, :, 2] + v1[:, :, 2] + v2[:, :, 2]) / 3.0
    inb = (cx >= -1.0) & (cx <= 1.0) & (cy >= -1.0) & (cy <= 1.0)
    valid = inb & (nz > 0.0)
    px = jnp.clip(jnp.round((cx + 1.0) * 0.5 * (W - 1)), 0, W - 1).astype(jnp.int32)
    py = jnp.clip(jnp.round((1.0 - cy) * 0.5 * (H - 1)), 0, H - 1).astype(jnp.int32)
    flat = py * W + px
    zkey = jnp.where(valid, -depth, -jnp.inf)

    featu = (uv[:, f0, 0] + uv[:, f1, 0] + uv[:, f2, 0]) / 3.0
    featv = (uv[:, f0, 1] + uv[:, f1, 1] + uv[:, f2, 1]) / 3.0

    pad = ((0, 0), (0, F_PAD - F))
    flat_p = jnp.pad(flat, pad).reshape(-1)
    zkey_p = jnp.pad(zkey, pad, constant_values=-jnp.inf).reshape(-1)
    fu_p = jnp.pad(featu, pad).reshape(-1)
    fv_p = jnp.pad(featv, pad).reshape(-1)
    zbuf = jax.vmap(lambda fl, zk: jnp.full((HW,), -jnp.inf, jnp.float32)
                    .at[fl].max(zk))(flat, zkey).reshape(-1)
    imgs_b = _accum_sc(flat_p, zkey_p, fu_p, fv_p, zbuf)[0]
    imgs = imgs_b.reshape(3, 4, HW)[:, dist_inds].reshape(-1)
    bases = jnp.broadcast_to((dist_inds.astype(jnp.int32) * TEXN)[:, None],
                             (4, 16)).reshape(-1)
    ren, prob, fg = _frag_sc(imgs, texture.reshape(-1), bases)

    imrender = ren.reshape(3, H, W).transpose(1, 2, 0)[None]
    improb = prob.reshape(H, W, 1)[None]
    fg_mask = fg.reshape(H, W, 1)[None]
    return (imrender, improb, n1_list, fg_mask)
